# baseline (speedup 1.0000x reference)
import functools

import jax
import jax.numpy as jnp
from jax import lax
from jax.experimental import pallas as pl
from jax.experimental.pallas import tpu as pltpu
from jax.experimental.pallas import tpu_sc as plsc

D = 64
G = 200
NBUF = 4


@functools.cache
def _make_format_kernel(V):
    info = plsc.get_sparse_core_info()
    NC, NS = info.num_cores, info.num_subcores
    NW = NC * NS
    n_blocks = V // 128
    tail = V - n_blocks * 128
    rounds = (n_blocks + NW - 1) // NW
    rounds += rounds % 2
    mesh = plsc.VectorSubcoreMesh(core_axis_name="c", subcore_axis_name="s")

    @functools.partial(
        pl.kernel,
        out_type=jax.ShapeDtypeStruct((V, 2 * D), jnp.float32),
        mesh=mesh,
        scratch_types=(
            [pltpu.VMEM((D, 128), jnp.float32) for _ in range(2)]
            + [pltpu.VMEM((128, 2 * D), jnp.float32) for _ in range(2)]
            + [pltpu.VMEM((D, 64), jnp.float32)]
            + [pltpu.SemaphoreType.DMA for _ in range(2)]
            + [pltpu.SemaphoreType.DMA for _ in range(2)]
        ),
        compiler_params=pltpu.CompilerParams(
            use_tc_tiling_on_sc=True, needs_layout_passes=False
        ),
    )
    def k(tt_hbm, scr_hbm, bin0, bin1, bout0, bout1, btail, si0, si1, so0, so1):
        bins = (bin0, bin1)
        bouts = (bout0, bout1)
        sin = (si0, si1)
        sout = (so0, so1)
        wid = lax.axis_index("s") * NC + lax.axis_index("c")

        def col_start(t):
            bb = t * NW + wid
            return bb, bb * 128

        iotas = [lax.iota(jnp.int32, 16) + 16 * g for g in range(8)]

        def transpose(bin_ref, bout_ref):
            @plsc.parallel_loop(0, D, unroll=16)
            def body(d):
                drot = (jnp.full((16,), 0, jnp.int32) + d + iotas[0]) & (D - 1)
                for g in range(8):
                    v = plsc.load_gather(bin_ref, [drot, iotas[g]])
                    plsc.store_scatter(bout_ref, [iotas[g], drot], v)

        def start_in(t, buf):
            bb, cs = col_start(t)
            @pl.when(bb < n_blocks)
            def _():
                pltpu.async_copy(
                    tt_hbm.at[:, pl.ds(cs, 128)], bins[buf], sin[buf]
                )

        def wait_in(t, buf):
            bb, cs = col_start(t)
            @pl.when(bb < n_blocks)
            def _():
                pltpu.make_async_copy(
                    tt_hbm.at[:, pl.ds(cs, 128)], bins[buf], sin[buf]
                ).wait()

        def start_out(t, buf):
            bb, cs = col_start(t)
            @pl.when(bb < n_blocks)
            def _():
                pltpu.async_copy(
                    bouts[buf], scr_hbm.at[pl.ds(cs, 128)], sout[buf]
                )

        def wait_out(t, buf):
            bb, cs = col_start(t)
            @pl.when(bb < n_blocks)
            def _():
                pltpu.make_async_copy(
                    bouts[buf], scr_hbm.at[pl.ds(cs, 128)], sout[buf]
                ).wait()

        start_in(0, 0)

        def outer(u, carry):
            for p in range(2):
                t = u * 2 + p
                wait_in(t, p)
                start_in(t + 1, 1 - p)

                @pl.when(t >= 2)
                def _():
                    wait_out(t - 2, p)

                bb, _ = col_start(t)

                @pl.when(bb < n_blocks)
                def _():
                    transpose(bins[p], bouts[p])
                start_out(t, p)
            return carry

        assert rounds % 2 == 0
        lax.fori_loop(0, rounds // 2, outer, 0)
        wait_out(rounds - 2, 0)
        wait_out(rounds - 1, 1)

        if tail:
            @pl.when(wid == 0)
            def _():
                pltpu.sync_copy(
                    tt_hbm.at[:, pl.ds(n_blocks * 128, tail)], btail
                )
                def tbody(d, carry):
                    dvec = jnp.full((16,), 0, jnp.int32) + d
                    for g in range(tail // 16):
                        v = plsc.load_gather(btail, [dvec, iotas[g]])
                        plsc.store_scatter(bout0, [iotas[g], dvec], v)
                    return carry
                lax.fori_loop(0, D, tbody, 0)
                pltpu.sync_copy(
                    bout0.at[pl.ds(0, tail)],
                    scr_hbm.at[pl.ds(n_blocks * 128, tail)],
                )

    return k


@functools.cache
def _make_gather_kernel(B, V):
    info = plsc.get_sparse_core_info()
    NC, NS = info.num_cores, info.num_subcores
    NW = NC * NS
    bpw = B // NW
    n_chunks = bpw // G
    assert bpw % G == 0 and n_chunks % NBUF == 0
    mesh = plsc.VectorSubcoreMesh(core_axis_name="c", subcore_axis_name="s")

    @functools.partial(
        pl.kernel,
        out_type=jax.ShapeDtypeStruct((B, 2 * D), jnp.float32),
        mesh=mesh,
        scratch_types=(
            [pltpu.VMEM((bpw,), jnp.int32)]
            + [pltpu.VMEM((G, 2 * D), jnp.float32) for _ in range(NBUF)]
            + [pltpu.SemaphoreType.DMA for _ in range(NBUF)]
        ),
        compiler_params=pltpu.CompilerParams(use_tc_tiling_on_sc=True),
    )
    def k(idx_hbm, scr_hbm, out_hbm, idx_v, *bufs_and_sems):
        bufs = bufs_and_sems[:NBUF]
        sems = bufs_and_sems[NBUF:]
        wid = lax.axis_index("s") * NC + lax.axis_index("c")
        base = wid * bpw
        pltpu.sync_copy(idx_hbm.at[pl.ds(base, bpw)], idx_v)

        def outer(t, carry):
            g0 = t * NBUF
            for b in range(NBUF):
                pltpu.async_copy(
                    scr_hbm.at[idx_v.at[pl.ds((g0 + b) * G, G)]],
                    bufs[b], sems[b],
                )
            for b in range(NBUF):
                pltpu.make_async_copy(
                    scr_hbm.at[idx_v.at[pl.ds((g0 + b) * G, G)]],
                    bufs[b], sems[b],
                ).wait()
                pltpu.sync_copy(
                    bufs[b], out_hbm.at[pl.ds(base + (g0 + b) * G, G)]
                )
            return carry

        lax.fori_loop(0, n_chunks // NBUF, outer, 0)

    return k


def kernel(x, table):
    R, C = x.shape
    V, Dd = table.shape
    B = R * C
    xf = x.reshape(B)
    scratch = _make_format_kernel(V)(table.T)
    out = _make_gather_kernel(B, V)(xf, scratch)
    return out[:, :Dd].reshape(R, C, Dd)


# speedup vs baseline: 1.0095x; 1.0095x over previous
import functools

import jax
import jax.numpy as jnp
from jax import lax
from jax.experimental import pallas as pl
from jax.experimental.pallas import tpu as pltpu
from jax.experimental.pallas import tpu_sc as plsc

D = 64
G = 200
NBUF = 4


@functools.cache
def _make_format_kernel(V):
    info = plsc.get_sparse_core_info()
    NC, NS = info.num_cores, info.num_subcores
    NW = NC * NS
    n_blocks = V // 128
    tail = V - n_blocks * 128
    rounds = (n_blocks + NW - 1) // NW
    rounds += rounds % 2
    mesh = plsc.VectorSubcoreMesh(core_axis_name="c", subcore_axis_name="s")

    @functools.partial(
        pl.kernel,
        out_type=jax.ShapeDtypeStruct((V, 2 * D), jnp.float32),
        mesh=mesh,
        scratch_types=(
            [pltpu.VMEM((D, 128), jnp.float32) for _ in range(2)]
            + [pltpu.VMEM((128, 2 * D), jnp.float32) for _ in range(2)]
            + [pltpu.VMEM((D, 64), jnp.float32)]
            + [pltpu.SemaphoreType.DMA for _ in range(2)]
            + [pltpu.SemaphoreType.DMA for _ in range(2)]
        ),
        compiler_params=pltpu.CompilerParams(
            use_tc_tiling_on_sc=True, needs_layout_passes=False
        ),
    )
    def k(tt_hbm, scr_hbm, bin0, bin1, bout0, bout1, btail, si0, si1, so0, so1):
        bins = (bin0, bin1)
        bouts = (bout0, bout1)
        sin = (si0, si1)
        sout = (so0, so1)
        wid = lax.axis_index("s") * NC + lax.axis_index("c")

        def col_start(t):
            bb = t * NW + wid
            return bb, bb * 128

        iotas = [lax.iota(jnp.int32, 16) + 16 * g for g in range(8)]

        def transpose(bin_ref, bout_ref):
            @plsc.parallel_loop(0, D, unroll=16)
            def body(d):
                drot = (jnp.full((16,), 0, jnp.int32) + d + iotas[0]) & (D - 1)
                for g in range(8):
                    v = plsc.load_gather(bin_ref, [drot, iotas[g]])
                    plsc.store_scatter(bout_ref, [iotas[g], drot], v)

        def start_in(t, buf):
            bb, cs = col_start(t)
            @pl.when(bb < n_blocks)
            def _():
                pltpu.async_copy(
                    tt_hbm.at[:, pl.ds(cs, 128)], bins[buf], sin[buf]
                )

        def wait_in(t, buf):
            bb, cs = col_start(t)
            @pl.when(bb < n_blocks)
            def _():
                pltpu.make_async_copy(
                    tt_hbm.at[:, pl.ds(cs, 128)], bins[buf], sin[buf]
                ).wait()

        def start_out(t, buf):
            bb, cs = col_start(t)
            @pl.when(bb < n_blocks)
            def _():
                pltpu.async_copy(
                    bouts[buf], scr_hbm.at[pl.ds(cs, 128)], sout[buf]
                )

        def wait_out(t, buf):
            bb, cs = col_start(t)
            @pl.when(bb < n_blocks)
            def _():
                pltpu.make_async_copy(
                    bouts[buf], scr_hbm.at[pl.ds(cs, 128)], sout[buf]
                ).wait()

        start_in(0, 0)

        def outer(u, carry):
            for p in range(2):
                t = u * 2 + p
                wait_in(t, p)
                start_in(t + 1, 1 - p)

                @pl.when(t >= 2)
                def _():
                    wait_out(t - 2, p)

                bb, _ = col_start(t)

                @pl.when(bb < n_blocks)
                def _():
                    transpose(bins[p], bouts[p])
                start_out(t, p)
            return carry

        assert rounds % 2 == 0
        lax.fori_loop(0, rounds // 2, outer, 0)
        wait_out(rounds - 2, 0)
        wait_out(rounds - 1, 1)

        if tail:
            @pl.when(wid == 0)
            def _():
                pltpu.sync_copy(
                    tt_hbm.at[:, pl.ds(n_blocks * 128, tail)], btail
                )
                def tbody(d, carry):
                    dvec = jnp.full((16,), 0, jnp.int32) + d
                    for g in range(tail // 16):
                        v = plsc.load_gather(btail, [dvec, iotas[g]])
                        plsc.store_scatter(bout0, [iotas[g], dvec], v)
                    return carry
                lax.fori_loop(0, D, tbody, 0)
                pltpu.sync_copy(
                    bout0.at[pl.ds(0, tail)],
                    scr_hbm.at[pl.ds(n_blocks * 128, tail)],
                )

    return k


@functools.cache
def _make_gather_kernel(B, V):
    info = plsc.get_sparse_core_info()
    NC, NS = info.num_cores, info.num_subcores
    NW = NC * NS
    bpw = B // NW
    n_chunks = bpw // G
    assert bpw % G == 0 and n_chunks % NBUF == 0
    mesh = plsc.VectorSubcoreMesh(core_axis_name="c", subcore_axis_name="s")

    @functools.partial(
        pl.kernel,
        out_type=jax.ShapeDtypeStruct((B, 2 * D), jnp.float32),
        mesh=mesh,
        scratch_types=(
            [pltpu.VMEM((bpw,), jnp.int32)]
            + [pltpu.VMEM((G, 2 * D), jnp.float32) for _ in range(NBUF)]
            + [pltpu.SemaphoreType.DMA for _ in range(2 * NBUF)]
        ),
        compiler_params=pltpu.CompilerParams(use_tc_tiling_on_sc=True),
    )
    def k(idx_hbm, scr_hbm, out_hbm, idx_v, *bufs_and_sems):
        bufs = bufs_and_sems[:NBUF]
        sems = bufs_and_sems[NBUF:2 * NBUF]
        wsems = bufs_and_sems[2 * NBUF:]
        wid = lax.axis_index("s") * NC + lax.axis_index("c")
        base = wid * bpw
        pltpu.sync_copy(idx_hbm.at[pl.ds(base, bpw)], idx_v)

        def outer(t, carry):
            g0 = t * NBUF
            for b in range(NBUF):
                @pl.when(t > 0)
                def _():
                    pltpu.make_async_copy(
                        bufs[b],
                        out_hbm.at[pl.ds(base + (g0 - NBUF + b) * G, G)],
                        wsems[b],
                    ).wait()
                pltpu.async_copy(
                    scr_hbm.at[idx_v.at[pl.ds((g0 + b) * G, G)]],
                    bufs[b], sems[b],
                )
            for b in range(NBUF):
                pltpu.make_async_copy(
                    scr_hbm.at[idx_v.at[pl.ds((g0 + b) * G, G)]],
                    bufs[b], sems[b],
                ).wait()
                pltpu.async_copy(
                    bufs[b], out_hbm.at[pl.ds(base + (g0 + b) * G, G)],
                    wsems[b],
                )
            return carry

        T = n_chunks // NBUF
        lax.fori_loop(0, T, outer, 0)
        for b in range(NBUF):
            pltpu.make_async_copy(
                bufs[b],
                out_hbm.at[pl.ds(base + (n_chunks - NBUF + b) * G, G)],
                wsems[b],
            ).wait()

    return k


def kernel(x, table):
    R, C = x.shape
    V, Dd = table.shape
    B = R * C
    xf = x.reshape(B)
    scratch = _make_format_kernel(V)(table.T)
    out = _make_gather_kernel(B, V)(xf, scratch)
    return out[:, :Dd].reshape(R, C, Dd)
